# baseline (speedup 1.0000x reference)
import functools

import jax
import jax.numpy as jnp
from jax import lax
from jax.experimental import pallas as pl
from jax.experimental.pallas import tpu as pltpu
from jax.experimental.pallas import tpu_sc as plsc

N = 65536
D = 64
K = 512
MAX_ITER = 10
TN = 2048
GRID = N // TN
SUB = 512


def _chunk_start(h):
    return jnp.where(h <= 7, 2240 * h, 15680 + 1920 * (h - 7))


def _distances(x, c):
    xc = jax.lax.dot_general(x.astype(jnp.bfloat16), c.astype(jnp.bfloat16),
                             (((1,), (1,)), ((), ())),
                             preferred_element_type=jnp.float32)
    x2 = jnp.sum(x * x, axis=1, keepdims=True)
    c2 = jnp.sum(c * c, axis=1)[None, :]
    return x2 - 2.0 * xc + c2


def _argmin_first(d):
    iota_k = jax.lax.broadcasted_iota(jnp.int32, d.shape, 1)
    mins = jnp.min(d, axis=1, keepdims=True)
    return jnp.min(jnp.where(d == mins, iota_k, K), axis=1).astype(jnp.int32)


def _assign_kernel(x_ref, c_ref, labels_ref, rank_ref, counts_ref, base_scr):
    i = pl.program_id(0)

    @pl.when(i == 0)
    def _():
        base_scr[...] = jnp.zeros((1, K), jnp.int32)

    x = x_ref[...]
    c = c_ref[...]
    d = _distances(x, c)
    labels = _argmin_first(d)
    iota_k = jax.lax.broadcasted_iota(jnp.int32, (TN, K), 1)
    oh = (iota_k == labels[:, None]).astype(jnp.int32)

    acc = oh
    sh = 1
    while sh < TN:
        shifted = jnp.concatenate(
            [jnp.zeros((sh, K), jnp.int32), acc[:TN - sh]], axis=0)
        acc = acc + shifted
        sh *= 2
    pre_excl = acc - oh

    base = base_scr[...]
    rank = jnp.sum((pre_excl + base) * oh, axis=1)
    labels_ref[...] = labels
    rank_ref[...] = rank
    tile_cnt = jnp.sum(oh, axis=0)

    @pl.when(i == 0)
    def _():
        counts_ref[...] = tile_cnt

    @pl.when(i != 0)
    def _():
        counts_ref[...] += tile_cnt

    base_scr[...] = base + tile_cnt[None, :]


def _assign(x, c):
    return pl.pallas_call(
        _assign_kernel,
        grid=(GRID,),
        in_specs=[
            pl.BlockSpec((TN, D), lambda i: (i, 0)),
            pl.BlockSpec((K, D), lambda i: (0, 0)),
        ],
        out_specs=[
            pl.BlockSpec((TN,), lambda i: (i,)),
            pl.BlockSpec((TN,), lambda i: (i,)),
            pl.BlockSpec((K,), lambda i: (0,)),
        ],
        out_shape=[
            jax.ShapeDtypeStruct((N,), jnp.int32),
            jax.ShapeDtypeStruct((N,), jnp.int32),
            jax.ShapeDtypeStruct((K,), jnp.int32),
        ],
        scratch_shapes=[pltpu.VMEM((1, K), jnp.int32)],
    )(x, c)


def _prep_kernel(counts_ref, labels_ref, rank_ref, p_ref, slab_ref):
    i = pl.program_id(0)
    cnt = counts_ref[...][None, :]
    acc = cnt
    sh = 1
    while sh < K:
        shifted = jnp.concatenate(
            [jnp.zeros((1, sh), jnp.int32), acc[:, :K - sh]], axis=1)
        acc = acc + shifted
        sh *= 2
    ends = acc
    starts = ends - cnt

    labels = labels_ref[...]
    rank = rank_ref[...]
    iota_k = jax.lax.broadcasted_iota(jnp.int32, (TN, K), 1)
    oh = iota_k == labels[:, None]
    p = jnp.sum(jnp.where(oh, starts, 0), axis=1) + rank
    p_ref[...] = p

    pos = i * TN + jax.lax.broadcasted_iota(jnp.int32, (TN, 1), 0)
    slab = jnp.sum((pos >= ends).astype(jnp.int32), axis=1)
    slab_ref[...] = slab


def _prep(counts, labels, rank):
    return pl.pallas_call(
        _prep_kernel,
        grid=(GRID,),
        in_specs=[
            pl.BlockSpec((K,), lambda i: (0,)),
            pl.BlockSpec((TN,), lambda i: (i,)),
            pl.BlockSpec((TN,), lambda i: (i,)),
        ],
        out_specs=[
            pl.BlockSpec((TN,), lambda i: (i,)),
            pl.BlockSpec((TN,), lambda i: (i,)),
        ],
        out_shape=[
            jax.ShapeDtypeStruct((N,), jnp.int32),
            jax.ShapeDtypeStruct((N,), jnp.int32),
        ],
    )(counts, labels, rank)


@functools.lru_cache(maxsize=1)
def _sc_kernels():
    mesh = plsc.VectorSubcoreMesh(core_axis_name="c", subcore_axis_name="s")

    @functools.partial(
        pl.kernel,
        out_type=jax.ShapeDtypeStruct((N, D), jnp.float32),
        mesh=mesh,
        compiler_params=pltpu.CompilerParams(use_tc_tiling_on_sc=False),
        scratch_types=[
            pltpu.VMEM((16, 128), jnp.int32),
            pltpu.VMEM((128, D), jnp.float32),
            pltpu.SemaphoreType.DMA,
        ],
    )
    def sc_scatter(x_hbm, p2d_hbm, xs_hbm, idx_v, rows_v, sem):
        wid = lax.axis_index("s") * 2 + lax.axis_index("c")
        base = wid * TN
        pltpu.sync_copy(p2d_hbm.at[pl.ds(wid * 16, 16)], idx_v)
        for j in range(16):
            pltpu.sync_copy(x_hbm.at[pl.ds(base + j * 128, 128)], rows_v)
            pltpu.async_copy(rows_v, xs_hbm.at[idx_v.at[j]], sem).wait()

    @functools.partial(
        pl.kernel,
        out_type=jax.ShapeDtypeStruct((32, K * D), jnp.float32),
        mesh=mesh,
        scratch_types=[
            pltpu.VMEM((K * D,), jnp.float32),
            pltpu.VMEM((SUB * D,), jnp.float32),
            pltpu.VMEM((SUB,), jnp.int32),
        ],
    )
    def sc_accum(xsf_hbm, slab_hbm, zeros_hbm, out_hbm, part_v, sub_v, lab_v):
        wid = lax.axis_index("s") * 2 + lax.axis_index("c")
        half = wid // 16
        h = wid % 16
        a0 = half * 32768 + _chunk_start(h)
        b0 = half * 32768 + jnp.where(h == 15, 32768, _chunk_start(h + 1))
        pltpu.sync_copy(zeros_hbm, part_v)
        for s in range(5):
            a = a0 + s * SUB

            @pl.when(a < b0)
            def _():
                a_c = jnp.minimum(a, N - SUB)
                pltpu.sync_copy(xsf_hbm.at[pl.ds(a_c * D, SUB * D)], sub_v)
                pltpu.sync_copy(slab_hbm.at[pl.ds(a_c, SUB)], lab_v)
                off = a - a_c
                r_hi = jnp.minimum(b0, a + SUB) - a

                def body(b, carry):
                    rb0 = b * 16 + off
                    lab_vec = lab_v[pl.ds(rb0, 16)]
                    for l in range(16):
                        lab = lab_vec[l]
                        rb = rb0 + l
                        for j in range(4):
                            sl_p = pl.ds(lab * D + j * 16, 16)
                            sl_x = pl.ds(rb * D + j * 16, 16)
                            part_v[sl_p] = part_v[sl_p] + sub_v[sl_x]
                    return carry

                lax.fori_loop(0, r_hi // 16, body, 0)

        pltpu.sync_copy(part_v, out_hbm.at[wid])

    return sc_scatter, sc_accum


def _update_kernel(parts_ref, counts_ref, c_ref, newc_ref):
    acc = parts_ref[0]
    for t in range(1, 32):
        acc = acc + parts_ref[t]
    cnt = counts_ref[...].astype(jnp.float32)
    newc = jnp.where(cnt[:, None] > 0.0,
                     acc / jnp.maximum(cnt, 1.0)[:, None],
                     c_ref[...])
    newc_ref[...] = newc


def _update(parts, counts, c):
    return pl.pallas_call(
        _update_kernel,
        out_shape=jax.ShapeDtypeStruct((K, D), jnp.float32),
    )(parts, counts, c)


def _assign_final_kernel(x_ref, c_ref, labels_ref, inertia_ref):
    i = pl.program_id(0)
    x = x_ref[...]
    c = c_ref[...]
    d = _distances(x, c)
    labels_ref[...] = _argmin_first(d)
    part = jnp.sum(jnp.min(d, axis=1)).reshape(1, 1)

    @pl.when(i == 0)
    def _():
        inertia_ref[...] = part

    @pl.when(i != 0)
    def _():
        inertia_ref[...] += part


def _assign_final(x, c):
    return pl.pallas_call(
        _assign_final_kernel,
        grid=(GRID,),
        in_specs=[
            pl.BlockSpec((TN, D), lambda i: (i, 0)),
            pl.BlockSpec((K, D), lambda i: (0, 0)),
        ],
        out_specs=[
            pl.BlockSpec((TN,), lambda i: (i,)),
            pl.BlockSpec((1, 1), lambda i: (0, 0)),
        ],
        out_shape=[
            jax.ShapeDtypeStruct((N,), jnp.int32),
            jax.ShapeDtypeStruct((1, 1), jnp.float32),
        ],
    )(x, c)


def kernel(x, centers):
    c0 = centers[0]
    zflat = jnp.zeros((K * D,), jnp.float32)

    sc_scatter, sc_accum = _sc_kernels()

    def step(c, _):
        labels, rank, counts = _assign(x, c)
        p, slab = _prep(counts, labels, rank)
        xs = sc_scatter(x, p.reshape(N // 128, 128))
        parts = sc_accum(xs.reshape(N * D), slab, zflat)
        c2 = _update(parts.reshape(32, K, D), counts, c)
        return c2, None

    c, _ = lax.scan(step, c0, None, length=MAX_ITER)
    labels, inertia = _assign_final(x, c)
    return labels, c, inertia.reshape(())
